# baseline (speedup 1.0000x reference)
import functools

import jax
import jax.numpy as jnp
from jax import lax
from jax.experimental import pallas as pl
from jax.experimental.pallas import tpu as pltpu
from jax.experimental.pallas import tpu_sc as plsc

_N = 10000
_E = 320000
_D = 128
_DO = 16

_NC = 2
_NS = 16
_NW = _NC * _NS
_C = 80
_GRP = 8
_EPAD = _NW * 10240 - _E
_TOTCHUNK = (_E + _EPAD) // _C
_CPP = _TOTCHUNK // _NS
_CH0 = 224
_CH1 = _CPP - _CH0
_NG0 = _CH0 // _GRP
_NG1 = _CH1 // _GRP
_RPT = 624
_RPT_LAST = _N - (_NS - 1) * _RPT


def _make_spmm(d):
    mesh = plsc.VectorSubcoreMesh(core_axis_name="c", subcore_axis_name="s")

    @functools.partial(
        pl.kernel,
        mesh=mesh,
        out_type=jax.ShapeDtypeStruct((_NC, _N, d), jnp.float32),
        scratch_types=[
            pltpu.VMEM((2, _GRP, _C), jnp.int32),
            pltpu.VMEM((2, _GRP, _C), jnp.int32),
            pltpu.VMEM((2, _GRP, _C), jnp.float32),
            pltpu.VMEM((_C, d), jnp.float32),
            pltpu.VMEM((_C, d), jnp.float32),
            pltpu.VMEM((_C, d), jnp.float32),
            pltpu.VMEM((_C, d), jnp.float32),
            pltpu.VMEM_SHARED((_N, d), jnp.float32),
            pltpu.SemaphoreType.DMA,
            pltpu.SemaphoreType.DMA,
            pltpu.SemaphoreType.DMA,
            pltpu.SemaphoreType.DMA,
            pltpu.SemaphoreType.DMA,
            pltpu.SemaphoreType.DMA,
            pltpu.SemaphoreType.DMA,
            pltpu.SemaphoreType.DMA,
            pltpu.SemaphoreType.DMA,
        ],
    )
    def spmm(z_hbm, src_hbm, dst_hbm, w_hbm, out_hbm,
             src_v, dst_v, w_v, rows0, rows1, rows2, rows3, acc_sh,
             sem_g0, sem_g1, sem_g2, sem_g3,
             sem_s0, sem_s1, sem_s2, sem_s3, sem_i):
        c = lax.axis_index("c")
        s = lax.axis_index("s")
        bchunk = pl.multiple_of(
            jnp.where(c == 0, s * _CH0, _NS * _CH0 + s * _CH1), 8)
        ngrp = jnp.where(c == 0, _NG0, _NG1)
        base = pl.multiple_of(s * _RPT, 8)
        def zrow(i, carry0):
            for cb in range(d // 16):
                rows0[i, pl.ds(cb * 16, 16)] = jnp.zeros((16,), jnp.float32)
            return carry0

        lax.fori_loop(0, _C, zrow, 0)
        nblk = _N // _C
        for k in range((nblk + _NS - 1) // _NS):
            b = s + k * _NS

            @pl.when(b < nblk)
            def _():
                zb = pl.multiple_of(b * _C, 8)
                pltpu.sync_copy(rows0, acc_sh.at[pl.ds(zb, _C)])
        pltpu.sync_copy(src_hbm.at[pl.ds(bchunk, _GRP)], src_v.at[0])
        pltpu.sync_copy(dst_hbm.at[pl.ds(bchunk, _GRP)], dst_v.at[0])
        pltpu.sync_copy(w_hbm.at[pl.ds(bchunk, _GRP)], w_v.at[0])
        plsc.subcore_barrier()

        @pl.when(ngrp > 0)
        def _():
            pltpu.make_async_copy(
                z_hbm.at[src_v.at[0, 0]], rows0, sem_g0).start()
            pltpu.make_async_copy(
                z_hbm.at[src_v.at[0, 1]], rows1, sem_g1).start()

        def grp_loop(g, carry):
            gb = jnp.bitwise_and(g, 1)
            gb1 = 1 - gb

            @pl.when(g < ngrp - 1)
            def _():
                nb = pl.multiple_of(bchunk + (g + 1) * _GRP, 8)
                pltpu.make_async_copy(
                    src_hbm.at[pl.ds(nb, _GRP)], src_v.at[gb1], sem_i
                ).start()
                pltpu.make_async_copy(
                    dst_hbm.at[pl.ds(nb, _GRP)], dst_v.at[gb1], sem_i
                ).start()
                pltpu.make_async_copy(
                    w_hbm.at[pl.ds(nb, _GRP)], w_v.at[gb1], sem_i
                ).start()

            rows_ring = (rows0, rows1, rows2, rows3)
            semg_ring = (sem_g0, sem_g1, sem_g2, sem_g3)
            sems_ring = (sem_s0, sem_s1, sem_s2, sem_s3)
            for k in range(_GRP):
                rows_b = rows_ring[k % 4]
                sem_gb = semg_ring[k % 4]
                sem_sb = sems_ring[k % 4]
                rows_n = rows_ring[(k + 2) % 4]
                sem_gn = semg_ring[(k + 2) % 4]
                sem_sn = sems_ring[(k + 2) % 4]

                pltpu.make_async_copy(
                    z_hbm.at[src_v.at[gb, k]], rows_b, sem_gb).wait()

                def _wait_prev():
                    pltpu.make_async_copy(
                        rows_n, acc_sh.at[dst_v.at[gb, k]], sem_sn).wait()
                if k < 2:
                    pl.when(g > 0)(_wait_prev)
                else:
                    _wait_prev()

                if k < _GRP - 2:
                    pltpu.make_async_copy(
                        z_hbm.at[src_v.at[gb, k + 2]], rows_n, sem_gn).start()
                else:
                    @pl.when(g < ngrp - 1)
                    def _():
                        if k == _GRP - 2:
                            pltpu.make_async_copy(
                                src_hbm.at[pl.ds(bchunk, _GRP)],
                                src_v.at[gb1], sem_i).wait()
                            pltpu.make_async_copy(
                                dst_hbm.at[pl.ds(bchunk, _GRP)],
                                dst_v.at[gb1], sem_i).wait()
                            pltpu.make_async_copy(
                                w_hbm.at[pl.ds(bchunk, _GRP)],
                                w_v.at[gb1], sem_i).wait()
                        pltpu.make_async_copy(
                            z_hbm.at[src_v.at[gb1, k - (_GRP - 2)]], rows_n,
                            sem_gn).start()

                @plsc.parallel_loop(0, _C // 16, unroll=2)
                def _(g16):
                    w16 = w_v[gb, k, pl.ds(g16 * 16, 16)]
                    for gg in range(16):
                        r = g16 * 16 + gg
                        sw = w16[gg]
                        for cb in range(d // 16):
                            sl = pl.ds(cb * 16, 16)
                            rows_b[r, sl] = rows_b[r, sl] * sw

                pltpu.async_copy(
                    rows_b, acc_sh.at[dst_v.at[gb, k]], sem_sb, add=True)
            return carry

        lax.fori_loop(0, ngrp, grp_loop, 0)

        @pl.when(ngrp > 0)
        def _():
            pltpu.make_async_copy(
                rows2, acc_sh.at[dst_v.at[0, 0]], sem_s2).wait()
            pltpu.make_async_copy(
                rows3, acc_sh.at[dst_v.at[0, 0]], sem_s3).wait()
        plsc.subcore_barrier()

        @pl.when(s < _NS - 1)
        def _():
            pltpu.sync_copy(acc_sh.at[pl.ds(base, _RPT)],
                            out_hbm.at[c, pl.ds(base, _RPT)])

        @pl.when(s == _NS - 1)
        def _():
            pltpu.sync_copy(acc_sh.at[pl.ds((_NS - 1) * _RPT, _RPT_LAST)],
                            out_hbm.at[c, pl.ds((_NS - 1) * _RPT, _RPT_LAST)])

    return spmm


_spmm128 = _make_spmm(_D)

_BLK = 1000
_G = _N // _BLK


def _lin_body(x_ref, w_ref, b_ref, o_ref):
    o_ref[...] = lax.dot_general(
        x_ref[...], w_ref[...], (((1,), (1,)), ((), ())),
        preferred_element_type=jnp.float32) + b_ref[...]


def _linear(x, w, b):
    dout = w.shape[0]
    return pl.pallas_call(
        _lin_body,
        grid=(_G,),
        in_specs=[
            pl.BlockSpec((_BLK, _D), lambda i: (i, 0)),
            pl.BlockSpec((dout, _D), lambda i: (0, 0)),
            pl.BlockSpec((1, dout), lambda i: (0, 0)),
        ],
        out_specs=pl.BlockSpec((_BLK, dout), lambda i: (i, 0)),
        out_shape=jax.ShapeDtypeStruct((_N, dout), jnp.float32),
    )(x, w, b.reshape(1, dout))


def _comb_body(p0_ref, p1_ref, hp_ref, t_ref, w_ref, b_ref, h_ref, z_ref):
    f = jnp.maximum(p0_ref[...] + p1_ref[...], 0.0)
    h = hp_ref[...] + t_ref[0, 0] * f
    h_ref[...] = h
    z_ref[...] = lax.dot_general(
        h, w_ref[...], (((1,), (1,)), ((), ())),
        preferred_element_type=jnp.float32) + b_ref[...]


def _combine_linear(p0, p1, hprev, t, w, b):
    dout = w.shape[0]
    return pl.pallas_call(
        _comb_body,
        grid=(_G,),
        in_specs=[
            pl.BlockSpec((_BLK, _D), lambda i: (i, 0)),
            pl.BlockSpec((_BLK, _D), lambda i: (i, 0)),
            pl.BlockSpec((_BLK, _D), lambda i: (i, 0)),
            pl.BlockSpec((1, 1), lambda i: (0, 0)),
            pl.BlockSpec((dout, _D), lambda i: (0, 0)),
            pl.BlockSpec((1, dout), lambda i: (0, 0)),
        ],
        out_specs=[
            pl.BlockSpec((_BLK, _D), lambda i: (i, 0)),
            pl.BlockSpec((_BLK, dout), lambda i: (i, 0)),
        ],
        out_shape=[
            jax.ShapeDtypeStruct((_N, _D), jnp.float32),
            jax.ShapeDtypeStruct((_N, dout), jnp.float32),
        ],
    )(p0, p1, hprev, t, w, b.reshape(1, dout))


def _add_body(a_ref, b_ref, o_ref):
    o_ref[...] = a_ref[..., :_DO] + b_ref[..., :_DO]


def _add2(a, b):
    return pl.pallas_call(
        _add_body,
        grid=(_G,),
        in_specs=[
            pl.BlockSpec((_BLK, _D), lambda i: (i, 0)),
            pl.BlockSpec((_BLK, _D), lambda i: (i, 0)),
        ],
        out_specs=pl.BlockSpec((_BLK, _DO), lambda i: (i, 0)),
        out_shape=jax.ShapeDtypeStruct((_N, _DO), jnp.float32),
    )(a, b)


def kernel(x, edge_index, edge_weight, W1, b1, Wm0, bm0, Wm1, bm1, W2, b2,
           time_step):
    ipad = jnp.zeros((_EPAD,), jnp.int32)
    src3 = jnp.concatenate([edge_index[1], ipad]).reshape(_TOTCHUNK, _C)
    dst3 = jnp.concatenate([edge_index[0], ipad]).reshape(_TOTCHUNK, _C)
    w3 = jnp.concatenate([edge_weight, jnp.zeros((_EPAD,), jnp.float32)]
                         ).reshape(_TOTCHUNK, _C)
    zero_h = jnp.zeros((_N, _D), jnp.float32)
    one = jnp.ones((1, 1), jnp.float32)
    t2 = time_step.reshape(1, 1)
    W2p = jnp.concatenate([W2, jnp.zeros((_D - _DO, _D), jnp.float32)])
    b2p = jnp.concatenate([b2, jnp.zeros((_D - _DO,), jnp.float32)])

    z1 = _linear(x, W1, b1)
    p = _spmm128(z1, src3, dst3, w3)
    h1, z2 = _combine_linear(p[0], p[1], zero_h, one, Wm0, bm0)
    p = _spmm128(z2, src3, dst3, w3)
    h2, z3 = _combine_linear(p[0], p[1], h1, t2, Wm1, bm1)
    p = _spmm128(z3, src3, dst3, w3)
    h3, z4 = _combine_linear(p[0], p[1], h2, t2, W2p, b2p)
    p4 = _spmm128(z4, src3, dst3, w3)
    return _add2(p4[0], p4[1])


# speedup vs baseline: 1.0704x; 1.0704x over previous
import functools

import jax
import jax.numpy as jnp
from jax import lax
from jax.experimental import pallas as pl
from jax.experimental.pallas import tpu as pltpu
from jax.experimental.pallas import tpu_sc as plsc

_N = 10000
_E = 320000
_D = 128
_DO = 16

_NC = 2
_NS = 16
_NW = _NC * _NS
_C = 80
_GRP = 8
_EPAD = _NW * 10240 - _E
_TOTCHUNK = (_E + _EPAD) // _C
_CPP = _TOTCHUNK // _NS
_CH0 = 240
_CH1 = _CPP - _CH0
_NG0 = _CH0 // _GRP
_NG1 = _CH1 // _GRP
_RPT = 624
_RPT_LAST = _N - (_NS - 1) * _RPT


def _make_spmm(d):
    mesh = plsc.VectorSubcoreMesh(core_axis_name="c", subcore_axis_name="s")

    @functools.partial(
        pl.kernel,
        mesh=mesh,
        out_type=jax.ShapeDtypeStruct((_NC, _N, d), jnp.float32),
        scratch_types=[
            pltpu.VMEM((2, _GRP, _C), jnp.int32),
            pltpu.VMEM((2, _GRP, _C), jnp.int32),
            pltpu.VMEM((2, _GRP, _C), jnp.float32),
            pltpu.VMEM((_C, d), jnp.float32),
            pltpu.VMEM((_C, d), jnp.float32),
            pltpu.VMEM((_C, d), jnp.float32),
            pltpu.VMEM((_C, d), jnp.float32),
            pltpu.VMEM_SHARED((_N, d), jnp.float32),
            pltpu.SemaphoreType.DMA,
            pltpu.SemaphoreType.DMA,
            pltpu.SemaphoreType.DMA,
            pltpu.SemaphoreType.DMA,
            pltpu.SemaphoreType.DMA,
            pltpu.SemaphoreType.DMA,
            pltpu.SemaphoreType.DMA,
            pltpu.SemaphoreType.DMA,
            pltpu.SemaphoreType.DMA,
        ],
    )
    def spmm(z_hbm, src_hbm, dst_hbm, w_hbm, out_hbm,
             src_v, dst_v, w_v, rows0, rows1, rows2, rows3, acc_sh,
             sem_g0, sem_g1, sem_g2, sem_g3,
             sem_s0, sem_s1, sem_s2, sem_s3, sem_i):
        c = lax.axis_index("c")
        s = lax.axis_index("s")
        bchunk = pl.multiple_of(
            jnp.where(c == 0, s * _CH0, _NS * _CH0 + s * _CH1), 8)
        ngrp = jnp.where(c == 0, _NG0, _NG1)
        base = pl.multiple_of(s * _RPT, 8)
        def zrow(i, carry0):
            for cb in range(d // 16):
                rows0[i, pl.ds(cb * 16, 16)] = jnp.zeros((16,), jnp.float32)
            return carry0

        lax.fori_loop(0, _C, zrow, 0)
        nblk = _N // _C
        for k in range((nblk + _NS - 1) // _NS):
            b = s + k * _NS

            @pl.when(b < nblk)
            def _():
                zb = pl.multiple_of(b * _C, 8)
                pltpu.sync_copy(rows0, acc_sh.at[pl.ds(zb, _C)])
        pltpu.sync_copy(src_hbm.at[pl.ds(bchunk, _GRP)], src_v.at[0])
        pltpu.sync_copy(dst_hbm.at[pl.ds(bchunk, _GRP)], dst_v.at[0])
        pltpu.sync_copy(w_hbm.at[pl.ds(bchunk, _GRP)], w_v.at[0])
        plsc.subcore_barrier()

        @pl.when(ngrp > 0)
        def _():
            pltpu.make_async_copy(
                z_hbm.at[src_v.at[0, 0]], rows0, sem_g0).start()
            pltpu.make_async_copy(
                z_hbm.at[src_v.at[0, 1]], rows1, sem_g1).start()

        def grp_loop(g, carry):
            gb = jnp.bitwise_and(g, 1)
            gb1 = 1 - gb

            @pl.when(g < ngrp - 1)
            def _():
                nb = pl.multiple_of(bchunk + (g + 1) * _GRP, 8)
                pltpu.make_async_copy(
                    src_hbm.at[pl.ds(nb, _GRP)], src_v.at[gb1], sem_i
                ).start()
                pltpu.make_async_copy(
                    dst_hbm.at[pl.ds(nb, _GRP)], dst_v.at[gb1], sem_i
                ).start()
                pltpu.make_async_copy(
                    w_hbm.at[pl.ds(nb, _GRP)], w_v.at[gb1], sem_i
                ).start()

            rows_ring = (rows0, rows1, rows2, rows3)
            semg_ring = (sem_g0, sem_g1, sem_g2, sem_g3)
            sems_ring = (sem_s0, sem_s1, sem_s2, sem_s3)
            for k in range(_GRP):
                rows_b = rows_ring[k % 4]
                sem_gb = semg_ring[k % 4]
                sem_sb = sems_ring[k % 4]
                rows_n = rows_ring[(k + 2) % 4]
                sem_gn = semg_ring[(k + 2) % 4]
                sem_sn = sems_ring[(k + 2) % 4]

                pltpu.make_async_copy(
                    z_hbm.at[src_v.at[gb, k]], rows_b, sem_gb).wait()

                def _wait_prev():
                    pltpu.make_async_copy(
                        rows_n, acc_sh.at[dst_v.at[gb, k]], sem_sn).wait()
                if k < 2:
                    pl.when(g > 0)(_wait_prev)
                else:
                    _wait_prev()

                if k < _GRP - 2:
                    pltpu.make_async_copy(
                        z_hbm.at[src_v.at[gb, k + 2]], rows_n, sem_gn).start()
                else:
                    @pl.when(g < ngrp - 1)
                    def _():
                        if k == _GRP - 2:
                            pltpu.make_async_copy(
                                src_hbm.at[pl.ds(bchunk, _GRP)],
                                src_v.at[gb1], sem_i).wait()
                            pltpu.make_async_copy(
                                dst_hbm.at[pl.ds(bchunk, _GRP)],
                                dst_v.at[gb1], sem_i).wait()
                            pltpu.make_async_copy(
                                w_hbm.at[pl.ds(bchunk, _GRP)],
                                w_v.at[gb1], sem_i).wait()
                        pltpu.make_async_copy(
                            z_hbm.at[src_v.at[gb1, k - (_GRP - 2)]], rows_n,
                            sem_gn).start()

                @plsc.parallel_loop(0, _C // 16, unroll=2)
                def _(g16):
                    w16 = w_v[gb, k, pl.ds(g16 * 16, 16)]
                    for gg in range(16):
                        r = g16 * 16 + gg
                        sw = w16[gg]
                        for cb in range(d // 16):
                            sl = pl.ds(cb * 16, 16)
                            rows_b[r, sl] = rows_b[r, sl] * sw

                pltpu.async_copy(
                    rows_b, acc_sh.at[dst_v.at[gb, k]], sem_sb, add=True)
            return carry

        lax.fori_loop(0, ngrp, grp_loop, 0)

        @pl.when(ngrp > 0)
        def _():
            pltpu.make_async_copy(
                rows2, acc_sh.at[dst_v.at[0, 0]], sem_s2).wait()
            pltpu.make_async_copy(
                rows3, acc_sh.at[dst_v.at[0, 0]], sem_s3).wait()
        plsc.subcore_barrier()

        @pl.when(s < _NS - 1)
        def _():
            pltpu.sync_copy(acc_sh.at[pl.ds(base, _RPT)],
                            out_hbm.at[c, pl.ds(base, _RPT)])

        @pl.when(s == _NS - 1)
        def _():
            pltpu.sync_copy(acc_sh.at[pl.ds((_NS - 1) * _RPT, _RPT_LAST)],
                            out_hbm.at[c, pl.ds((_NS - 1) * _RPT, _RPT_LAST)])

    return spmm


_spmm128 = _make_spmm(_D)

_BLK = 1000
_G = _N // _BLK


def _lin_body(x_ref, w_ref, b_ref, o_ref):
    o_ref[...] = lax.dot_general(
        x_ref[...], w_ref[...], (((1,), (1,)), ((), ())),
        preferred_element_type=jnp.float32) + b_ref[...]


def _linear(x, w, b):
    dout = w.shape[0]
    return pl.pallas_call(
        _lin_body,
        grid=(_G,),
        in_specs=[
            pl.BlockSpec((_BLK, _D), lambda i: (i, 0)),
            pl.BlockSpec((dout, _D), lambda i: (0, 0)),
            pl.BlockSpec((1, dout), lambda i: (0, 0)),
        ],
        out_specs=pl.BlockSpec((_BLK, dout), lambda i: (i, 0)),
        out_shape=jax.ShapeDtypeStruct((_N, dout), jnp.float32),
    )(x, w, b.reshape(1, dout))


def _comb_body(p0_ref, p1_ref, hp_ref, t_ref, w_ref, b_ref, h_ref, z_ref):
    f = jnp.maximum(p0_ref[...] + p1_ref[...], 0.0)
    h = hp_ref[...] + t_ref[0, 0] * f
    h_ref[...] = h
    z_ref[...] = lax.dot_general(
        h, w_ref[...], (((1,), (1,)), ((), ())),
        preferred_element_type=jnp.float32) + b_ref[...]


def _combine_linear(p0, p1, hprev, t, w, b):
    dout = w.shape[0]
    return pl.pallas_call(
        _comb_body,
        grid=(_G,),
        in_specs=[
            pl.BlockSpec((_BLK, _D), lambda i: (i, 0)),
            pl.BlockSpec((_BLK, _D), lambda i: (i, 0)),
            pl.BlockSpec((_BLK, _D), lambda i: (i, 0)),
            pl.BlockSpec((1, 1), lambda i: (0, 0)),
            pl.BlockSpec((dout, _D), lambda i: (0, 0)),
            pl.BlockSpec((1, dout), lambda i: (0, 0)),
        ],
        out_specs=[
            pl.BlockSpec((_BLK, _D), lambda i: (i, 0)),
            pl.BlockSpec((_BLK, dout), lambda i: (i, 0)),
        ],
        out_shape=[
            jax.ShapeDtypeStruct((_N, _D), jnp.float32),
            jax.ShapeDtypeStruct((_N, dout), jnp.float32),
        ],
    )(p0, p1, hprev, t, w, b.reshape(1, dout))


def _add_body(a_ref, b_ref, o_ref):
    o_ref[...] = a_ref[..., :_DO] + b_ref[..., :_DO]


def _add2(a, b):
    return pl.pallas_call(
        _add_body,
        grid=(_G,),
        in_specs=[
            pl.BlockSpec((_BLK, _D), lambda i: (i, 0)),
            pl.BlockSpec((_BLK, _D), lambda i: (i, 0)),
        ],
        out_specs=pl.BlockSpec((_BLK, _DO), lambda i: (i, 0)),
        out_shape=jax.ShapeDtypeStruct((_N, _DO), jnp.float32),
    )(a, b)


def kernel(x, edge_index, edge_weight, W1, b1, Wm0, bm0, Wm1, bm1, W2, b2,
           time_step):
    ipad = jnp.zeros((_EPAD,), jnp.int32)
    src3 = jnp.concatenate([edge_index[1], ipad]).reshape(_TOTCHUNK, _C)
    dst3 = jnp.concatenate([edge_index[0], ipad]).reshape(_TOTCHUNK, _C)
    w3 = jnp.concatenate([edge_weight, jnp.zeros((_EPAD,), jnp.float32)]
                         ).reshape(_TOTCHUNK, _C)
    zero_h = jnp.zeros((_N, _D), jnp.float32)
    one = jnp.ones((1, 1), jnp.float32)
    t2 = time_step.reshape(1, 1)
    W2p = jnp.concatenate([W2, jnp.zeros((_D - _DO, _D), jnp.float32)])
    b2p = jnp.concatenate([b2, jnp.zeros((_D - _DO,), jnp.float32)])

    z1 = _linear(x, W1, b1)
    p = _spmm128(z1, src3, dst3, w3)
    h1, z2 = _combine_linear(p[0], p[1], zero_h, one, Wm0, bm0)
    p = _spmm128(z2, src3, dst3, w3)
    h2, z3 = _combine_linear(p[0], p[1], h1, t2, Wm1, bm1)
    p = _spmm128(z3, src3, dst3, w3)
    h3, z4 = _combine_linear(p[0], p[1], h2, t2, W2p, b2p)
    p4 = _spmm128(z4, src3, dst3, w3)
    return _add2(p4[0], p4[1])


# speedup vs baseline: 1.0713x; 1.0008x over previous
import functools

import jax
import jax.numpy as jnp
from jax import lax
from jax.experimental import pallas as pl
from jax.experimental.pallas import tpu as pltpu
from jax.experimental.pallas import tpu_sc as plsc

_N = 10000
_E = 320000
_D = 128
_DO = 16

_NC = 2
_NS = 16
_NW = _NC * _NS
_C = 80
_GRP = 8
_EPAD = _NW * 10240 - _E
_TOTCHUNK = (_E + _EPAD) // _C
_CPP = _TOTCHUNK // _NS
_CH0 = 240
_CH1 = _CPP - _CH0
_NG0 = _CH0 // _GRP
_NG1 = _CH1 // _GRP
_RPT = 624
_RPT_LAST = _N - (_NS - 1) * _RPT


def _make_spmm(d):
    mesh = plsc.VectorSubcoreMesh(core_axis_name="c", subcore_axis_name="s")

    @functools.partial(
        pl.kernel,
        mesh=mesh,
        out_type=jax.ShapeDtypeStruct((_NC, _N, d), jnp.float32),
        scratch_types=[
            pltpu.VMEM((2, _GRP, _C), jnp.int32),
            pltpu.VMEM((2, _GRP, _C), jnp.int32),
            pltpu.VMEM((2, _GRP, _C), jnp.float32),
            pltpu.VMEM((_C, d), jnp.float32),
            pltpu.VMEM((_C, d), jnp.float32),
            pltpu.VMEM((_C, d), jnp.float32),
            pltpu.VMEM((_C, d), jnp.float32),
            pltpu.VMEM_SHARED((_N, d), jnp.float32),
            pltpu.SemaphoreType.DMA,
            pltpu.SemaphoreType.DMA,
            pltpu.SemaphoreType.DMA,
            pltpu.SemaphoreType.DMA,
            pltpu.SemaphoreType.DMA,
            pltpu.SemaphoreType.DMA,
            pltpu.SemaphoreType.DMA,
            pltpu.SemaphoreType.DMA,
            pltpu.SemaphoreType.DMA,
        ],
    )
    def spmm(z_hbm, src_hbm, dst_hbm, w_hbm, out_hbm,
             src_v, dst_v, w_v, rows0, rows1, rows2, rows3, acc_sh,
             sem_g0, sem_g1, sem_g2, sem_g3,
             sem_s0, sem_s1, sem_s2, sem_s3, sem_i):
        c = lax.axis_index("c")
        s = lax.axis_index("s")
        bchunk = pl.multiple_of(
            jnp.where(c == 0, s * _CH0, _NS * _CH0 + s * _CH1), 8)
        ngrp = jnp.where(c == 0, _NG0, _NG1)
        base = pl.multiple_of(s * _RPT, 8)
        def zrow(i, carry0):
            for cb in range(d // 16):
                rows0[i, pl.ds(cb * 16, 16)] = jnp.zeros((16,), jnp.float32)
            return carry0

        lax.fori_loop(0, _C, zrow, 0)
        nblk = _N // _C
        for k in range((nblk + _NS - 1) // _NS):
            b = s + k * _NS

            @pl.when(b < nblk)
            def _():
                zb = pl.multiple_of(b * _C, 8)
                pltpu.sync_copy(rows0, acc_sh.at[pl.ds(zb, _C)])
        pltpu.sync_copy(src_hbm.at[pl.ds(bchunk, _GRP)], src_v.at[0])
        pltpu.sync_copy(dst_hbm.at[pl.ds(bchunk, _GRP)], dst_v.at[0])
        pltpu.sync_copy(w_hbm.at[pl.ds(bchunk, _GRP)], w_v.at[0])
        plsc.subcore_barrier()

        @pl.when(ngrp > 0)
        def _():
            pltpu.make_async_copy(
                z_hbm.at[src_v.at[0, 0]], rows0, sem_g0).start()
            pltpu.make_async_copy(
                z_hbm.at[src_v.at[0, 1]], rows1, sem_g1).start()

        def grp_loop(g, carry):
            gb = jnp.bitwise_and(g, 1)
            gb1 = 1 - gb

            @pl.when(g < ngrp - 1)
            def _():
                nb = pl.multiple_of(bchunk + (g + 1) * _GRP, 8)
                pltpu.make_async_copy(
                    src_hbm.at[pl.ds(nb, _GRP)], src_v.at[gb1], sem_i
                ).start()
                pltpu.make_async_copy(
                    dst_hbm.at[pl.ds(nb, _GRP)], dst_v.at[gb1], sem_i
                ).start()
                pltpu.make_async_copy(
                    w_hbm.at[pl.ds(nb, _GRP)], w_v.at[gb1], sem_i
                ).start()

            rows_ring = (rows0, rows1, rows2, rows3)
            semg_ring = (sem_g0, sem_g1, sem_g2, sem_g3)
            sems_ring = (sem_s0, sem_s1, sem_s2, sem_s3)
            for k in range(_GRP):
                rows_b = rows_ring[k % 4]
                sem_gb = semg_ring[k % 4]
                sem_sb = sems_ring[k % 4]
                rows_n = rows_ring[(k + 2) % 4]
                sem_gn = semg_ring[(k + 2) % 4]
                sem_sn = sems_ring[(k + 2) % 4]

                pltpu.make_async_copy(
                    z_hbm.at[src_v.at[gb, k]], rows_b, sem_gb).wait()

                def _wait_prev():
                    pltpu.make_async_copy(
                        rows_n, acc_sh.at[dst_v.at[gb, k]], sem_sn).wait()
                if k < 2:
                    pl.when(g > 0)(_wait_prev)
                else:
                    _wait_prev()

                if k < _GRP - 2:
                    pltpu.make_async_copy(
                        z_hbm.at[src_v.at[gb, k + 2]], rows_n, sem_gn).start()
                else:
                    @pl.when(g < ngrp - 1)
                    def _():
                        if k == _GRP - 2:
                            pltpu.make_async_copy(
                                src_hbm.at[pl.ds(bchunk, _GRP)],
                                src_v.at[gb1], sem_i).wait()
                            pltpu.make_async_copy(
                                dst_hbm.at[pl.ds(bchunk, _GRP)],
                                dst_v.at[gb1], sem_i).wait()
                            pltpu.make_async_copy(
                                w_hbm.at[pl.ds(bchunk, _GRP)],
                                w_v.at[gb1], sem_i).wait()
                        pltpu.make_async_copy(
                            z_hbm.at[src_v.at[gb1, k - (_GRP - 2)]], rows_n,
                            sem_gn).start()

                @plsc.parallel_loop(0, _C // 16, unroll=2)
                def _(g16):
                    w16 = w_v[gb, k, pl.ds(g16 * 16, 16)]
                    for gg in range(16):
                        r = g16 * 16 + gg
                        sw = w16[gg]
                        for cb in range(d // 16):
                            sl = pl.ds(cb * 16, 16)
                            rows_b[r, sl] = rows_b[r, sl] * sw

                pltpu.async_copy(
                    rows_b, acc_sh.at[dst_v.at[gb, k]], sem_sb, add=True)
            return carry

        lax.fori_loop(0, ngrp, grp_loop, 0)

        @pl.when(ngrp > 0)
        def _():
            pltpu.make_async_copy(
                rows2, acc_sh.at[dst_v.at[0, 0]], sem_s2).wait()
            pltpu.make_async_copy(
                rows3, acc_sh.at[dst_v.at[0, 0]], sem_s3).wait()
        plsc.subcore_barrier()

        @pl.when(s < _NS - 1)
        def _():
            pltpu.sync_copy(acc_sh.at[pl.ds(base, _RPT)],
                            out_hbm.at[c, pl.ds(base, _RPT)])

        @pl.when(s == _NS - 1)
        def _():
            pltpu.sync_copy(acc_sh.at[pl.ds((_NS - 1) * _RPT, _RPT_LAST)],
                            out_hbm.at[c, pl.ds((_NS - 1) * _RPT, _RPT_LAST)])

    return spmm


_spmm_cache = {}


def _spmm128(*args):
    if _D not in _spmm_cache:
        _spmm_cache[_D] = _make_spmm(_D)
    return _spmm_cache[_D](*args)


_BLK = 1000
_G = _N // _BLK


def _lin_body(x_ref, w_ref, b_ref, o_ref):
    o_ref[...] = lax.dot_general(
        x_ref[...], w_ref[...], (((1,), (1,)), ((), ())),
        preferred_element_type=jnp.float32) + b_ref[...]


def _linear(x, w, b):
    dout = w.shape[0]
    return pl.pallas_call(
        _lin_body,
        grid=(_G,),
        in_specs=[
            pl.BlockSpec((_BLK, _D), lambda i: (i, 0)),
            pl.BlockSpec((dout, _D), lambda i: (0, 0)),
            pl.BlockSpec((1, dout), lambda i: (0, 0)),
        ],
        out_specs=pl.BlockSpec((_BLK, dout), lambda i: (i, 0)),
        out_shape=jax.ShapeDtypeStruct((_N, dout), jnp.float32),
    )(x, w, b.reshape(1, dout))


def _comb_body(p0_ref, p1_ref, hp_ref, t_ref, w_ref, b_ref, h_ref, z_ref):
    f = jnp.maximum(p0_ref[...] + p1_ref[...], 0.0)
    h = hp_ref[...] + t_ref[0, 0] * f
    h_ref[...] = h
    z_ref[...] = lax.dot_general(
        h, w_ref[...], (((1,), (1,)), ((), ())),
        preferred_element_type=jnp.float32) + b_ref[...]


def _combine_linear(p0, p1, hprev, t, w, b):
    dout = w.shape[0]
    return pl.pallas_call(
        _comb_body,
        grid=(_G,),
        in_specs=[
            pl.BlockSpec((_BLK, _D), lambda i: (i, 0)),
            pl.BlockSpec((_BLK, _D), lambda i: (i, 0)),
            pl.BlockSpec((_BLK, _D), lambda i: (i, 0)),
            pl.BlockSpec((1, 1), lambda i: (0, 0)),
            pl.BlockSpec((dout, _D), lambda i: (0, 0)),
            pl.BlockSpec((1, dout), lambda i: (0, 0)),
        ],
        out_specs=[
            pl.BlockSpec((_BLK, _D), lambda i: (i, 0)),
            pl.BlockSpec((_BLK, dout), lambda i: (i, 0)),
        ],
        out_shape=[
            jax.ShapeDtypeStruct((_N, _D), jnp.float32),
            jax.ShapeDtypeStruct((_N, dout), jnp.float32),
        ],
    )(p0, p1, hprev, t, w, b.reshape(1, dout))


def _add_body(a_ref, b_ref, o_ref):
    o_ref[...] = a_ref[..., :_DO] + b_ref[..., :_DO]


def _add2(a, b):
    return pl.pallas_call(
        _add_body,
        grid=(_G,),
        in_specs=[
            pl.BlockSpec((_BLK, _D), lambda i: (i, 0)),
            pl.BlockSpec((_BLK, _D), lambda i: (i, 0)),
        ],
        out_specs=pl.BlockSpec((_BLK, _DO), lambda i: (i, 0)),
        out_shape=jax.ShapeDtypeStruct((_N, _DO), jnp.float32),
    )(a, b)


def kernel(x, edge_index, edge_weight, W1, b1, Wm0, bm0, Wm1, bm1, W2, b2,
           time_step):
    ipad = jnp.zeros((_EPAD,), jnp.int32)
    src3 = jnp.concatenate([edge_index[1], ipad]).reshape(_TOTCHUNK, _C)
    dst3 = jnp.concatenate([edge_index[0], ipad]).reshape(_TOTCHUNK, _C)
    w3 = jnp.concatenate([edge_weight, jnp.zeros((_EPAD,), jnp.float32)]
                         ).reshape(_TOTCHUNK, _C)
    zero_h = jnp.zeros((_N, _D), jnp.float32)
    one = jnp.ones((1, 1), jnp.float32)
    t2 = time_step.reshape(1, 1)
    W2p = jnp.concatenate([W2, jnp.zeros((_D - _DO, _D), jnp.float32)])
    b2p = jnp.concatenate([b2, jnp.zeros((_D - _DO,), jnp.float32)])

    z1 = _linear(x, W1, b1)
    p = _spmm128(z1, src3, dst3, w3)
    h1, z2 = _combine_linear(p[0], p[1], zero_h, one, Wm0, bm0)
    p = _spmm128(z2, src3, dst3, w3)
    h2, z3 = _combine_linear(p[0], p[1], h1, t2, Wm1, bm1)
    p = _spmm128(z3, src3, dst3, w3)
    h3, z4 = _combine_linear(p[0], p[1], h2, t2, W2p, b2p)
    p4 = _spmm128(z4, src3, dst3, w3)
    return _add2(p4[0], p4[1])
